# baseline (speedup 1.0000x reference)
import functools

import jax
import jax.numpy as jnp
from jax.experimental import pallas as pl
from jax.experimental.pallas import tpu as pltpu

D_MODEL = 768
D_FF = 512
NUM_EXPERTS = 64
TOP_K = 2
T = 2048
NPAIR = T * TOP_K
TM = 32
NT = 192
NP = NT * TM
RC = 128
NRC = NPAIR // RC


WBITS = 20
WSCALE = float(1 << WBITS)
WMASK = (1 << WBITS) - 1


def _route_kernel(x_ref, gw_ref, slot_ref, pk_ref, ts_ref, tc_ref):
    x = x_ref[...]
    logits = jnp.dot(x, gw_ref[...], preferred_element_type=jnp.float32)
    iota = jax.lax.broadcasted_iota(jnp.int32, (T, NUM_EXPERTS), 1)
    m1 = jnp.max(logits, axis=1, keepdims=True)
    a1 = jnp.min(jnp.where(logits == m1, iota, NUM_EXPERTS), axis=1, keepdims=True)
    l2 = jnp.where(iota == a1, -jnp.inf, logits)
    m2 = jnp.max(l2, axis=1, keepdims=True)
    a2 = jnp.min(jnp.where(l2 == m2, iota, NUM_EXPERTS), axis=1, keepdims=True)
    eexp = jnp.exp(m2 - m1)
    w1c = 1.0 / (1.0 + eexp)
    w2c = eexp * w1c
    tokcol = jax.lax.broadcasted_iota(jnp.int32, (T, 1), 0)
    f1 = jnp.minimum((w1c * WSCALE).astype(jnp.int32), WMASK)
    f2 = jnp.minimum((w2c * WSCALE).astype(jnp.int32), WMASK)
    pk_ref[...] = jnp.concatenate(
        [(tokcol << WBITS) | f1, (tokcol << WBITS) | f2], axis=0)

    ohA = (iota == a1).astype(jnp.float32)
    ohB = (iota == a2).astype(jnp.float32)
    oh = jnp.concatenate([ohA, ohB], axis=0)

    ri = jax.lax.broadcasted_iota(jnp.int32, (RC, RC), 0)
    ci = jax.lax.broadcasted_iota(jnp.int32, (RC, RC), 1)
    lexc = (ci < ri).astype(jnp.float32)
    base = jnp.zeros((1, NUM_EXPERTS), jnp.float32)
    ranks = []
    for c in range(NRC):
        blk = oh[c * RC:(c + 1) * RC]
        ranks.append(jnp.dot(lexc, blk, preferred_element_type=jnp.float32) + base)
        base = base + jnp.sum(blk, axis=0, keepdims=True)
    rank_mat = jnp.concatenate(ranks, axis=0)
    counts = base

    tiles = jnp.ceil(counts * (1.0 / TM))
    ei = jax.lax.broadcasted_iota(jnp.int32, (NUM_EXPERTS, NUM_EXPERTS), 0)
    ej = jax.lax.broadcasted_iota(jnp.int32, (NUM_EXPERTS, NUM_EXPERTS), 1)
    uincl = (ei <= ej).astype(jnp.float32)
    cumt = jnp.dot(tiles, uincl, preferred_element_type=jnp.float32)
    tstart = cumt - tiles
    poff = tstart * TM
    slotf = jnp.sum(oh * (rank_mat + poff), axis=1, keepdims=True)
    slot_ref[...] = slotf.astype(jnp.int32)
    ts_ref[...] = tstart.astype(jnp.int32)
    tc_ref[...] = tiles.astype(jnp.int32)


def _moe_kernel(tstart_ref, tcnt_ref, src_ref, x_ref, pw_ref, w1_ref, w2_ref,
                out_ref, xg_ref):
    e = pl.program_id(0)

    @pl.when(e == 0)
    def _init():
        out_ref[...] = jnp.zeros_like(out_ref)

    t0 = tstart_ref[e]

    def _chunk(t, carry):
        base = (t0 + t) * TM
        for r in range(TM):
            s = src_ref[base + r] >> WBITS
            xg_ref[r:r + 1, :] = x_ref[pl.ds(s, 1), :]
        h = jnp.dot(xg_ref[...], w1_ref[0], preferred_element_type=jnp.float32,
                    precision=jax.lax.Precision.DEFAULT)
        h = h * jax.nn.sigmoid(h)
        y = jnp.dot(h, w2_ref[0], preferred_element_type=jnp.float32,
                    precision=jax.lax.Precision.DEFAULT)
        pv = pw_ref[pl.ds(t0 + t, 1)][0]
        y = y * ((pv & WMASK).astype(jnp.float32) * (1.0 / WSCALE))
        for r in range(TM):
            s = src_ref[base + r] >> WBITS
            out_ref[pl.ds(s, 1), :] += y[r:r + 1, :]
        return carry

    jax.lax.fori_loop(0, tcnt_ref[e], _chunk, 0)


@functools.partial(jax.jit, static_argnames=("interpret",))
def _run(inputs, gate_w, w1, w2, interpret=False):
    x = inputs.reshape(-1, D_MODEL)

    slot, packed, ts, tc = pl.pallas_call(
        _route_kernel,
        out_shape=[jax.ShapeDtypeStruct((NPAIR, 1), jnp.int32),
                   jax.ShapeDtypeStruct((NPAIR, 1), jnp.int32),
                   jax.ShapeDtypeStruct((1, NUM_EXPERTS), jnp.int32),
                   jax.ShapeDtypeStruct((1, NUM_EXPERTS), jnp.int32)],
        interpret=interpret,
    )(x, gate_w)

    srcpk = jnp.zeros(NP, jnp.int32).at[slot.reshape(-1)].set(packed.reshape(-1))
    tstart = ts.reshape(-1)
    tcnt = tc.reshape(-1)

    grid_spec = pltpu.PrefetchScalarGridSpec(
        num_scalar_prefetch=3,
        grid=(NUM_EXPERTS,),
        in_specs=[
            pl.BlockSpec((T, D_MODEL), lambda e, tsr, tcr, src: (0, 0)),
            pl.BlockSpec((NT, TM, 1), lambda e, tsr, tcr, src: (0, 0, 0)),
            pl.BlockSpec((1, D_MODEL, D_FF), lambda e, tsr, tcr, src: (e, 0, 0)),
            pl.BlockSpec((1, D_FF, D_MODEL), lambda e, tsr, tcr, src: (e, 0, 0)),
        ],
        out_specs=pl.BlockSpec((T, D_MODEL), lambda e, tsr, tcr, src: (0, 0)),
        scratch_shapes=[pltpu.VMEM((TM, D_MODEL), jnp.float32)],
    )
    out = pl.pallas_call(
        _moe_kernel,
        grid_spec=grid_spec,
        out_shape=jax.ShapeDtypeStruct((T, D_MODEL), jnp.float32),
        interpret=interpret,
    )(tstart, tcnt, srcpk, x, srcpk.reshape(NT, TM, 1), w1, w2)
    return out.reshape(inputs.shape)


def kernel(inputs, gate_w, w1, w2):
    return _run(inputs, gate_w, w1, w2)


# speedup vs baseline: 1.0318x; 1.0318x over previous
import functools

import jax
import jax.numpy as jnp
from jax.experimental import pallas as pl
from jax.experimental.pallas import tpu as pltpu

D_MODEL = 768
D_FF = 512
NUM_EXPERTS = 64
TOP_K = 2
T = 2048
NPAIR = T * TOP_K
TM = 64
NT = 128
NP = NT * TM
RC = 128
NRC = NPAIR // RC


WBITS = 20
WSCALE = float(1 << WBITS)
WMASK = (1 << WBITS) - 1


def _route_kernel(x_ref, gw_ref, slot_ref, pk_ref, ts_ref, tc_ref):
    x = x_ref[...]
    logits = jnp.dot(x, gw_ref[...], preferred_element_type=jnp.float32)
    iota = jax.lax.broadcasted_iota(jnp.int32, (T, NUM_EXPERTS), 1)
    m1 = jnp.max(logits, axis=1, keepdims=True)
    a1 = jnp.min(jnp.where(logits == m1, iota, NUM_EXPERTS), axis=1, keepdims=True)
    l2 = jnp.where(iota == a1, -jnp.inf, logits)
    m2 = jnp.max(l2, axis=1, keepdims=True)
    a2 = jnp.min(jnp.where(l2 == m2, iota, NUM_EXPERTS), axis=1, keepdims=True)
    eexp = jnp.exp(m2 - m1)
    w1c = 1.0 / (1.0 + eexp)
    w2c = eexp * w1c
    tokcol = jax.lax.broadcasted_iota(jnp.int32, (T, 1), 0)
    f1 = jnp.minimum((w1c * WSCALE).astype(jnp.int32), WMASK)
    f2 = jnp.minimum((w2c * WSCALE).astype(jnp.int32), WMASK)
    pk_ref[...] = jnp.concatenate(
        [(tokcol << WBITS) | f1, (tokcol << WBITS) | f2], axis=0)

    ohA = (iota == a1).astype(jnp.float32)
    ohB = (iota == a2).astype(jnp.float32)
    oh = jnp.concatenate([ohA, ohB], axis=0)

    ri = jax.lax.broadcasted_iota(jnp.int32, (RC, RC), 0)
    ci = jax.lax.broadcasted_iota(jnp.int32, (RC, RC), 1)
    lexc = (ci < ri).astype(jnp.float32)
    base = jnp.zeros((1, NUM_EXPERTS), jnp.float32)
    ranks = []
    for c in range(NRC):
        blk = oh[c * RC:(c + 1) * RC]
        ranks.append(jnp.dot(lexc, blk, preferred_element_type=jnp.float32) + base)
        base = base + jnp.sum(blk, axis=0, keepdims=True)
    rank_mat = jnp.concatenate(ranks, axis=0)
    counts = base

    tiles = jnp.ceil(counts * (1.0 / TM))
    ei = jax.lax.broadcasted_iota(jnp.int32, (NUM_EXPERTS, NUM_EXPERTS), 0)
    ej = jax.lax.broadcasted_iota(jnp.int32, (NUM_EXPERTS, NUM_EXPERTS), 1)
    uincl = (ei <= ej).astype(jnp.float32)
    cumt = jnp.dot(tiles, uincl, preferred_element_type=jnp.float32)
    tstart = cumt - tiles
    poff = tstart * TM
    slotf = jnp.sum(oh * (rank_mat + poff), axis=1, keepdims=True)
    slot_ref[...] = slotf.astype(jnp.int32)
    ts_ref[...] = tstart.astype(jnp.int32)
    tc_ref[...] = tiles.astype(jnp.int32)


def _moe_kernel(tstart_ref, tcnt_ref, src_ref, x_ref, pw_ref, w1_ref, w2_ref,
                out_ref, xg_ref):
    e = pl.program_id(0)

    @pl.when(e == 0)
    def _init():
        out_ref[...] = jnp.zeros_like(out_ref)

    t0 = tstart_ref[e]

    def _chunk(t, carry):
        base = (t0 + t) * TM
        for r in range(TM):
            s = src_ref[base + r] >> WBITS
            xg_ref[r:r + 1, :] = x_ref[pl.ds(s, 1), :]
        h = jnp.dot(xg_ref[...], w1_ref[0], preferred_element_type=jnp.float32,
                    precision=jax.lax.Precision.DEFAULT)
        h = h * jax.nn.sigmoid(h)
        y = jnp.dot(h, w2_ref[0], preferred_element_type=jnp.float32,
                    precision=jax.lax.Precision.DEFAULT)
        pv = pw_ref[pl.ds(t0 + t, 1)][0]
        y = y * ((pv & WMASK).astype(jnp.float32) * (1.0 / WSCALE))
        for r in range(TM):
            s = src_ref[base + r] >> WBITS
            out_ref[pl.ds(s, 1), :] += y[r:r + 1, :]
        return carry

    jax.lax.fori_loop(0, tcnt_ref[e], _chunk, 0)


@functools.partial(jax.jit, static_argnames=("interpret",))
def _run(inputs, gate_w, w1, w2, interpret=False):
    x = inputs.reshape(-1, D_MODEL)

    slot, packed, ts, tc = pl.pallas_call(
        _route_kernel,
        out_shape=[jax.ShapeDtypeStruct((NPAIR, 1), jnp.int32),
                   jax.ShapeDtypeStruct((NPAIR, 1), jnp.int32),
                   jax.ShapeDtypeStruct((1, NUM_EXPERTS), jnp.int32),
                   jax.ShapeDtypeStruct((1, NUM_EXPERTS), jnp.int32)],
        interpret=interpret,
    )(x, gate_w)

    srcpk = jnp.zeros(NP, jnp.int32).at[slot.reshape(-1)].set(packed.reshape(-1))
    tstart = ts.reshape(-1)
    tcnt = tc.reshape(-1)

    grid_spec = pltpu.PrefetchScalarGridSpec(
        num_scalar_prefetch=3,
        grid=(NUM_EXPERTS,),
        in_specs=[
            pl.BlockSpec((T, D_MODEL), lambda e, tsr, tcr, src: (0, 0)),
            pl.BlockSpec((NT, TM, 1), lambda e, tsr, tcr, src: (0, 0, 0)),
            pl.BlockSpec((1, D_MODEL, D_FF), lambda e, tsr, tcr, src: (e, 0, 0)),
            pl.BlockSpec((1, D_FF, D_MODEL), lambda e, tsr, tcr, src: (e, 0, 0)),
        ],
        out_specs=pl.BlockSpec((T, D_MODEL), lambda e, tsr, tcr, src: (0, 0)),
        scratch_shapes=[pltpu.VMEM((TM, D_MODEL), jnp.float32)],
    )
    out = pl.pallas_call(
        _moe_kernel,
        grid_spec=grid_spec,
        out_shape=jax.ShapeDtypeStruct((T, D_MODEL), jnp.float32),
        interpret=interpret,
    )(tstart, tcnt, srcpk, x, srcpk.reshape(NT, TM, 1), w1, w2)
    return out.reshape(inputs.shape)


def kernel(inputs, gate_w, w1, w2):
    return _run(inputs, gate_w, w1, w2)
